# Initial kernel scaffold; baseline (speedup 1.0000x reference)
import jax
import jax.numpy as jnp
from jax.experimental import pallas as pl


def kernel(images, predictions_0, predictions_1, predictions_2):
    raise NotImplementedError("write your pallas kernel here")


# speedup vs baseline: 1.1828x; 1.1828x over previous
import jax
import jax.numpy as jnp
from jax.experimental import pallas as pl
from jax.experimental.pallas import tpu as pltpu

_CLASSES = 80
_IOU_THRESH = 0.5
_MAX_DET = 100
_PRE_NMS_TOPK = 256


def _make_decode_body(level_shapes, img_h):
    sizes = [h * w for (h, w) in level_shapes]
    b0 = sizes[0]
    b1 = sizes[0] + sizes[1]
    ntot = sum(sizes)
    w0, w1, w2 = level_shapes[0][1], level_shapes[1][1], level_shapes[2][1]
    s0 = img_h / float(level_shapes[0][0])
    s1 = img_h / float(level_shapes[1][0])
    s2 = img_h / float(level_shapes[2][0])

    def body(pred_ref, scores_ref, boxes_ref):
        p = pred_ref[0]
        a = jax.lax.broadcasted_iota(jnp.int32, (ntot, 1), 0)
        lvl1 = a >= b0
        lvl2 = a >= b1
        gx = jnp.where(lvl2, (a - b1) % w2,
                       jnp.where(lvl1, (a - b0) % w1, a % w0))
        gy = jnp.where(lvl2, (a - b1) // w2,
                       jnp.where(lvl1, (a - b0) // w1, a // w0))
        s = jnp.where(lvl2, jnp.float32(s2),
                      jnp.where(lvl1, jnp.float32(s1), jnp.float32(s0)))
        x1 = (p[:, 0:1] + gx.astype(jnp.float32)) * s
        y1 = (p[:, 1:2] + gy.astype(jnp.float32)) * s
        w = jnp.exp(p[:, 2:3]) * s
        h = jnp.exp(p[:, 3:4]) * s
        boxes_ref[0] = jnp.concatenate([x1, y1, x1 + w, y1 + h], axis=1)
        conf = jax.nn.sigmoid(p[:, 4:5])
        probs = jax.nn.sigmoid(p[:, 5:5 + _CLASSES])
        scores_ref[0] = conf * probs

    return body


def _nms_body(boxes_ref, cls_ref, sc_ref, rows_ref, masked_ref, supp_ref):
    k = _PRE_NMS_TOPK
    b = boxes_ref[0]
    cls_r = cls_ref[0]
    sc_r = sc_ref[0]
    x1c, y1c, x2c, y2c = b[:, 0:1], b[:, 1:2], b[:, 2:3], b[:, 3:4]
    x1r = jnp.transpose(x1c)
    y1r = jnp.transpose(y1c)
    x2r = jnp.transpose(x2c)
    y2r = jnp.transpose(y2c)
    area_c = jnp.maximum(x2c - x1c, 0.0) * jnp.maximum(y2c - y1c, 0.0)
    area_r = jnp.transpose(area_c)
    xx1 = jnp.maximum(x1c, x1r)
    yy1 = jnp.maximum(y1c, y1r)
    xx2 = jnp.minimum(x2c, x2r)
    yy2 = jnp.minimum(y2c, y2r)
    inter = jnp.maximum(xx2 - xx1, 0.0) * jnp.maximum(yy2 - yy1, 0.0)
    union = area_c + area_r - inter
    iou = inter / jnp.maximum(union, 1e-8)
    same = jnp.transpose(cls_r) == cls_r
    li = jax.lax.broadcasted_iota(jnp.int32, (k, k), 0)
    lj = jax.lax.broadcasted_iota(jnp.int32, (k, k), 1)
    supp_ref[:, :] = jnp.where(same & (li < lj) & (iou > _IOU_THRESH),
                               jnp.float32(1.0), jnp.float32(0.0))
    lane = jax.lax.broadcasted_iota(jnp.int32, (1, k), 1)

    def body(i, keep):
        row = supp_ref[pl.ds(i, 1), :]
        ki = jnp.max(jnp.where(lane == i, keep, 0.0))
        return keep * (1.0 - ki * row)

    keep = jax.lax.fori_loop(0, k, body, jnp.ones((1, k), jnp.float32))
    masked_ref[0] = sc_r * keep
    rows = jnp.concatenate(
        [b, jnp.transpose(cls_r), jnp.transpose(sc_r)], axis=1)
    rows_ref[0] = rows * jnp.transpose(keep)


def kernel(images, predictions_0, predictions_1, predictions_2):
    preds = [predictions_0, predictions_1, predictions_2]
    batch = preds[0].shape[0]
    ch = 5 + _CLASSES
    level_shapes = [(p.shape[1], p.shape[2]) for p in preds]
    ntot = sum(h * w for (h, w) in level_shapes)
    img_h = float(images.shape[1])
    k = _PRE_NMS_TOPK

    flat = jnp.concatenate([p.reshape(batch, -1, ch) for p in preds], axis=1)

    f32 = jnp.float32
    scores, boxes = pl.pallas_call(
        _make_decode_body(level_shapes, img_h),
        grid=(batch,),
        in_specs=[pl.BlockSpec((1, ntot, ch), lambda i: (i, 0, 0))],
        out_specs=[
            pl.BlockSpec((1, ntot, _CLASSES), lambda i: (i, 0, 0)),
            pl.BlockSpec((1, ntot, 4), lambda i: (i, 0, 0)),
        ],
        out_shape=[
            jax.ShapeDtypeStruct((batch, ntot, _CLASSES), f32),
            jax.ShapeDtypeStruct((batch, ntot, 4), f32),
        ],
    )(flat)

    flat_scores = scores.reshape(batch, ntot * _CLASSES)
    top_s, idx = jax.lax.top_k(flat_scores, k)
    anchor = idx // _CLASSES
    cls = (idx % _CLASSES).astype(f32)
    cand_boxes = jnp.take_along_axis(boxes, anchor[..., None], axis=1)

    rows6, masked = pl.pallas_call(
        _nms_body,
        grid=(batch,),
        in_specs=[
            pl.BlockSpec((1, k, 4), lambda i: (i, 0, 0)),
            pl.BlockSpec((1, 1, k), lambda i: (i, 0, 0)),
            pl.BlockSpec((1, 1, k), lambda i: (i, 0, 0)),
        ],
        out_specs=[
            pl.BlockSpec((1, k, 6), lambda i: (i, 0, 0)),
            pl.BlockSpec((1, 1, k), lambda i: (i, 0, 0)),
        ],
        out_shape=[
            jax.ShapeDtypeStruct((batch, k, 6), f32),
            jax.ShapeDtypeStruct((batch, 1, k), f32),
        ],
        scratch_shapes=[pltpu.VMEM((k, k), f32)],
    )(cand_boxes, cls[:, None, :], top_s[:, None, :])

    _, i2 = jax.lax.top_k(masked[:, 0, :], _MAX_DET)
    return jnp.take_along_axis(rows6, i2[..., None], axis=1)
